# baseline (speedup 1.0000x reference)
import functools

import jax
import jax.numpy as jnp
from jax import lax
from jax.experimental import pallas as pl
from jax.experimental.pallas import tpu as pltpu
from jax.experimental.pallas import tpu_sc as plsc

N = 10000
E = 320000
F = 128
EPS = 1e-5

NC = 2
NS = 16
NW = NC * NS
L = 16

NPAD = 10240
NCHK = NPAD // NS

CH = 128
NCHUNK = 80
TOTCH = NW * NCHUNK
EPW = NCHUNK * CH
EP = NW * EPW

NCH0 = 80
NCH1 = 2 * NCHUNK - NCH0

ISQ = float(1.0 / (1.0 + EPS) ** 0.5)


@functools.lru_cache(maxsize=1)
def _sc_kernels():
    mesh = plsc.VectorSubcoreMesh(
        core_axis_name="c", subcore_axis_name="s",
        num_cores=NC, num_subcores=NS,
    )

    @functools.partial(
        pl.kernel,
        out_type=jax.ShapeDtypeStruct((NC, NPAD), jnp.float32),
        mesh=mesh,
        compiler_params=pltpu.CompilerParams(needs_layout_passes=False),
        scratch_types=[
            pltpu.VMEM((NCHUNK, CH), jnp.int32),
            pltpu.VMEM((NCHUNK, CH), jnp.float32),
            pltpu.VMEM((NCHK,), jnp.float32),
            pltpu.VMEM_SHARED((NPAD,), jnp.float32),
        ],
    )
    def deg_kernel(col_b, ew_b, deg_out, eidx, evalf, tmp, deg_sh):
        cid = lax.axis_index("c")
        sid = lax.axis_index("s")
        wid = sid * NC + cid
        nbase = sid * NCHK

        def _z(i, c):
            tmp[pl.ds(i * L, L)] = jnp.zeros((L,), jnp.float32)
            return c
        lax.fori_loop(0, NCHK // L, _z, 0)
        pltpu.sync_copy(tmp, deg_sh.at[pl.ds(nbase, NCHK)])
        plsc.subcore_barrier()

        pltpu.sync_copy(col_b.at[wid], eidx)
        pltpu.sync_copy(ew_b.at[wid], evalf)

        def _deg(j, c):
            pltpu.sync_copy(evalf.at[j], deg_sh.at[eidx.at[j]], add=True)
            return c
        lax.fori_loop(0, NCHUNK, _deg, 0)
        plsc.subcore_barrier()
        pltpu.sync_copy(deg_sh.at[pl.ds(nbase, NCHK)],
                        deg_out.at[cid, pl.ds(nbase, NCHK)])

    @functools.partial(
        pl.kernel,
        out_type=jax.ShapeDtypeStruct((NW, NCHUNK, CH), jnp.float32),
        mesh=mesh,
        compiler_params=pltpu.CompilerParams(needs_layout_passes=False),
        scratch_types=[
            pltpu.VMEM((NCHUNK, CH), jnp.int32),
            pltpu.VMEM((NCHUNK, CH), jnp.int32),
            pltpu.VMEM((NCHUNK, CH), jnp.float32),
            pltpu.VMEM((NCHUNK, CH), jnp.float32),
            pltpu.VMEM((NPAD,), jnp.float32),
        ],
    )
    def norm_kernel(row_b, col_b, ew_b, dinv, norm_out,
                    ridx, cidx, ewb, nbuf, dinv_loc):
        cid = lax.axis_index("c")
        sid = lax.axis_index("s")
        wid = sid * NC + cid

        pltpu.sync_copy(dinv, dinv_loc)
        pltpu.sync_copy(row_b.at[wid], ridx)
        pltpu.sync_copy(col_b.at[wid], cidx)
        pltpu.sync_copy(ew_b.at[wid], ewb)

        def _nrm(j, c):
            for k in range(CH // L):
                sl = pl.ds(k * L, L)
                ri = ridx[j, sl]
                ci = cidx[j, sl]
                w = ewb[j, sl]
                dv = plsc.load_gather(dinv_loc, [ri])
                dc = plsc.load_gather(dinv_loc, [ci])
                nbuf[j, sl] = dv * w * dc
            return c
        lax.fori_loop(0, NCHUNK, _nrm, 0)
        pltpu.sync_copy(nbuf, norm_out.at[wid])

    @functools.partial(
        pl.kernel,
        out_type=jax.ShapeDtypeStruct((NC, NPAD, F), jnp.float32),
        mesh=mesh,
        compiler_params=pltpu.CompilerParams(needs_layout_passes=False),
        scratch_types=[
            pltpu.VMEM((4, CH), jnp.int32),
            pltpu.VMEM((4, CH), jnp.int32),
            pltpu.VMEM((4, CH), jnp.float32),
            pltpu.VMEM((CH, F), jnp.float32),
            pltpu.VMEM((CH, F), jnp.float32),
            pltpu.VMEM_SHARED((NPAD, F), jnp.float32),
            pltpu.SemaphoreType.DMA,
            pltpu.SemaphoreType.DMA,
            pltpu.SemaphoreType.DMA,
            pltpu.SemaphoreType.DMA,
            pltpu.SemaphoreType.DMA,
            pltpu.SemaphoreType.DMA,
            pltpu.SemaphoreType.DMA,
            pltpu.SemaphoreType.DMA,
        ],
    )
    def scatter_kernel(h, row_f, col_f, norm_f, part,
                       ridxr, cidxr, nrmr, buf0, buf1, acc_sh,
                       sg0, sg1, si0, si1, si2, si3, ss0, ss1):
        cid = lax.axis_index("c")
        sid = lax.axis_index("s")
        nbase = sid * NCHK
        bufs = (buf0, buf1)
        sg = (sg0, sg1)
        si = (si0, si1, si2, si3)
        ss = (ss0, ss1)

        def _zr(r, c):
            for k in range(F // L):
                buf0[r, pl.ds(k * L, L)] = jnp.zeros((L,), jnp.float32)
            return c
        lax.fori_loop(0, CH, _zr, 0)
        for b in range(NCHK // CH):
            pltpu.sync_copy(buf0, acc_sh.at[pl.ds(nbase + b * CH, CH)])
        plsc.subcore_barrier()

        def fire_idx(q, s):
            pltpu.async_copy(row_f.at[q], ridxr.at[s], si[s])
            pltpu.async_copy(col_f.at[q], cidxr.at[s], si[s])
            pltpu.async_copy(norm_f.at[q], nrmr.at[s], si[s])

        def wait_idx(s):
            pltpu.make_async_copy(row_f.at[0], ridxr.at[s], si[s]).wait()
            pltpu.make_async_copy(col_f.at[0], cidxr.at[s], si[s]).wait()
            pltpu.make_async_copy(norm_f.at[0], nrmr.at[s], si[s]).wait()

        def fire_gather(rs, bs):
            pltpu.async_copy(h.at[ridxr.at[rs]], bufs[bs], sg[bs])

        def wait_gather(rs, bs):
            pltpu.make_async_copy(h.at[ridxr.at[rs]], bufs[bs],
                                  sg[bs]).wait()

        def fire_scatter(rs, bs):
            pltpu.async_copy(bufs[bs], acc_sh.at[cidxr.at[rs]], ss[bs],
                             add=True)

        def wait_scatter(rs, bs):
            pltpu.make_async_copy(bufs[bs], acc_sh.at[cidxr.at[rs]],
                                  ss[bs]).wait()

        def scale(rs, bs):
            r = bufs[bs]

            def _sce(e, c2):
                se = jnp.full((L,), rs, jnp.int32)
                ee = jnp.full((L,), e, jnp.int32)
                m = plsc.load_gather(nrmr, [se, ee])
                for k in range(F // L):
                    sl = pl.ds(k * L, L)
                    r[e, sl] = r[e, sl] * m
                return c2
            lax.fori_loop(0, CH, _sce, 0, unroll=2)

        def step(q, rs, wait_prev, fire1):
            bs = rs % 2
            nbs = 1 - bs
            wait_gather(rs, bs)
            if wait_prev:
                wait_scatter((rs + 3) % 4, nbs)
            if fire1:
                fire_idx(q + 1, (rs + 1) % 4)
                wait_idx((rs + 1) % 4)
                fire_gather((rs + 1) % 4, nbs)
            scale(rs, bs)
            fire_scatter(rs, bs)

        def pipeline(base, n):
            fire_idx(base, 0)
            wait_idx(0)
            fire_gather(0, 0)
            step(base, 0, wait_prev=False, fire1=True)
            step(base + 1, 1, wait_prev=True, fire1=True)

            def _quad(jj, c):
                q = base + 4 * jj + 2
                step(q, 2, wait_prev=True, fire1=True)
                step(q + 1, 3, wait_prev=True, fire1=True)
                step(q + 2, 0, wait_prev=True, fire1=True)
                step(q + 3, 1, wait_prev=True, fire1=True)
                return c
            lax.fori_loop(0, (n - 4) // 4, _quad, 0)

            step(base + n - 2, 2, wait_prev=True, fire1=True)
            step(base + n - 1, 3, wait_prev=True, fire1=False)
            wait_scatter(3, 1)

        @pl.when(cid == 0)
        def _():
            pipeline(sid * NCH0, NCH0)

        @pl.when(cid == 1)
        def _():
            pipeline(NS * NCH0 + sid * NCH1, NCH1)

        plsc.subcore_barrier()

        pltpu.sync_copy(acc_sh.at[pl.ds(nbase, NCHK)],
                        part.at[cid, pl.ds(nbase, NCHK)])

    return deg_kernel, norm_kernel, scatter_kernel


_BLK = 640


def _dinv_body(dp_ref, dinv_ref, sn_ref):
    d = dp_ref[0] + dp_ref[1] + 1.0
    r = lax.rsqrt(d)
    dinv_ref[...] = r
    sn_ref[...] = r * r


def _tc_dinv(deg_parts):
    m = NPAD // F
    return pl.pallas_call(
        _dinv_body,
        out_shape=(
            jax.ShapeDtypeStruct((m, F), jnp.float32),
            jax.ShapeDtypeStruct((m, F), jnp.float32),
        ),
    )(deg_parts)


def _mm_body(x_ref, w_ref, o_ref):
    o_ref[...] = jnp.dot(x_ref[...], w_ref[...],
                         preferred_element_type=jnp.float32)


def _tc_matmul(x, w):
    return pl.pallas_call(
        _mm_body,
        grid=(NPAD // _BLK,),
        in_specs=[
            pl.BlockSpec((_BLK, F), lambda i: (i, 0)),
            pl.BlockSpec((F, F), lambda i: (0, 0)),
        ],
        out_specs=pl.BlockSpec((_BLK, F), lambda i: (i, 0)),
        out_shape=jax.ShapeDtypeStruct((NPAD, F), jnp.float32),
    )(x, w)


def _comb_body(p0_ref, p1_ref, hm_ref, sn_ref, b_ref, g_ref, bt_ref, w2_ref,
               h1_ref, h2m_ref):
    s = p0_ref[...] + p1_ref[...] + hm_ref[...] * sn_ref[...]
    h = jnp.maximum(s + b_ref[...], 0.0)
    h = h * (g_ref[...] * ISQ) + bt_ref[...]
    h1_ref[...] = h
    h2m_ref[...] = jnp.dot(h, w2_ref[...], preferred_element_type=jnp.float32)


def _tc_combine_mm(p0, p1, hm, sn, b, g, bt, w2):
    vec = pl.BlockSpec((1, F), lambda i: (0, 0))
    blk = pl.BlockSpec((_BLK, F), lambda i: (i, 0))
    col = pl.BlockSpec((_BLK, 1), lambda i: (i, 0))
    return pl.pallas_call(
        _comb_body,
        grid=(NPAD // _BLK,),
        in_specs=[blk, blk, blk, col, vec, vec, vec,
                  pl.BlockSpec((F, F), lambda i: (0, 0))],
        out_specs=(blk, blk),
        out_shape=(
            jax.ShapeDtypeStruct((NPAD, F), jnp.float32),
            jax.ShapeDtypeStruct((NPAD, F), jnp.float32),
        ),
    )(p0, p1, hm, sn, b.reshape(1, F), g.reshape(1, F), bt.reshape(1, F), w2)


def _head_body(p0_ref, p1_ref, hm_ref, sn_ref, b_ref, g_ref, bt_ref,
               x_ref, h1_ref,
               f1a_ref, f1b_ref, f1c_ref, fb1_ref, f2w_ref, f2b_ref, o_ref):
    s = p0_ref[...] + p1_ref[...] + hm_ref[...] * sn_ref[...]
    h2 = jnp.maximum(s + b_ref[...], 0.0)
    h2 = h2 * (g_ref[...] * ISQ) + bt_ref[...]
    z = (jnp.dot(x_ref[...], f1a_ref[...], preferred_element_type=jnp.float32)
         + jnp.dot(h1_ref[...], f1b_ref[...],
                   preferred_element_type=jnp.float32)
         + jnp.dot(h2, f1c_ref[...], preferred_element_type=jnp.float32))
    z = jnp.maximum(z + fb1_ref[...], 0.0)
    y = jnp.dot(z, f2w_ref[...], preferred_element_type=jnp.float32)
    o_ref[...] = jnp.maximum(y + f2b_ref[...], 0.0)


def _tc_head(p0, p1, hm, sn, b2, g2, bt2, x, h1, f1a, f1b, f1c, fb1,
             f2w, f2b):
    vec = pl.BlockSpec((1, F), lambda i: (0, 0))
    blk = pl.BlockSpec((_BLK, F), lambda i: (i, 0))
    col = pl.BlockSpec((_BLK, 1), lambda i: (i, 0))
    wspec = pl.BlockSpec((F, F), lambda i: (0, 0))
    return pl.pallas_call(
        _head_body,
        grid=(NPAD // _BLK,),
        in_specs=[blk, blk, blk, col, vec, vec, vec, blk, blk,
                  wspec, wspec, wspec,
                  vec, pl.BlockSpec((F, 1), lambda i: (0, 0)),
                  pl.BlockSpec((1, 1), lambda i: (0, 0))],
        out_specs=col,
        out_shape=jax.ShapeDtypeStruct((NPAD, 1), jnp.float32),
    )(p0, p1, hm, sn, b2.reshape(1, F), g2.reshape(1, F), bt2.reshape(1, F),
      x, h1, f1a, f1b, f1c, fb1.reshape(1, F), f2w, f2b.reshape(1, 1))


def kernel(x, adj_indices, adj_values, W1, b1, bn1_gamma, bn1_beta,
           W2, b2, bn2_gamma, bn2_beta, fc1_W, fc1_b, fc2_W, fc2_b):
    deg_kernel, norm_kernel, scatter_kernel = _sc_kernels()
    ipad = jnp.zeros((EP - E,), jnp.int32)
    row_p = jnp.concatenate([adj_indices[0], ipad])
    col_p = jnp.concatenate([adj_indices[1], ipad])
    ew_p = jnp.concatenate([adj_values, jnp.zeros((EP - E,), jnp.float32)])
    row_b = row_p.reshape(NW, NCHUNK, CH)
    col_b = col_p.reshape(NW, NCHUNK, CH)
    ew_b = ew_p.reshape(NW, NCHUNK, CH)
    row_f = row_p.reshape(TOTCH, CH)
    col_f = col_p.reshape(TOTCH, CH)
    x_pad = jnp.concatenate([x, jnp.zeros((NPAD - N, F), jnp.float32)])

    deg_parts = deg_kernel(col_b, ew_b)
    dinv2d, sn2d = _tc_dinv(deg_parts.reshape(NC, NPAD // F, F))
    norm_b = norm_kernel(row_b, col_b, ew_b, dinv2d.reshape(NPAD))
    norm_f = norm_b.reshape(TOTCH, CH)
    sn = sn2d.reshape(NPAD, 1)

    h1m = _tc_matmul(x_pad, W1)
    part1 = scatter_kernel(h1m, row_f, col_f, norm_f)
    h1, h2m = _tc_combine_mm(part1[0], part1[1], h1m, sn,
                             b1, bn1_gamma, bn1_beta, W2)
    part2 = scatter_kernel(h2m, row_f, col_f, norm_f)
    y = _tc_head(part2[0], part2[1], h2m, sn, b2, bn2_gamma, bn2_beta,
                 x_pad, h1, fc1_W[:F], fc1_W[F:2 * F], fc1_W[2 * F:],
                 fc1_b, fc2_W, fc2_b)
    return y.reshape(-1)[:N]


# speedup vs baseline: 1.1701x; 1.1701x over previous
import functools

import jax
import jax.numpy as jnp
from jax import lax
from jax.experimental import pallas as pl
from jax.experimental.pallas import tpu as pltpu
from jax.experimental.pallas import tpu_sc as plsc

N = 10000
E = 320000
F = 128
EPS = 1e-5

NC = 2
NS = 16
NW = NC * NS
L = 16

NPAD = 10240
NCHK = NPAD // NS

CH = 128
NCHUNK = 80
TOTCH = NW * NCHUNK
EPW = NCHUNK * CH
EP = NW * EPW

NCH0 = 120
NCH1 = 2 * NCHUNK - NCH0

ISQ = float(1.0 / (1.0 + EPS) ** 0.5)


@functools.lru_cache(maxsize=1)
def _sc_kernels():
    mesh = plsc.VectorSubcoreMesh(
        core_axis_name="c", subcore_axis_name="s",
        num_cores=NC, num_subcores=NS,
    )

    @functools.partial(
        pl.kernel,
        out_type=jax.ShapeDtypeStruct((NC, NPAD), jnp.float32),
        mesh=mesh,
        compiler_params=pltpu.CompilerParams(needs_layout_passes=False),
        scratch_types=[
            pltpu.VMEM((NCHUNK, CH), jnp.int32),
            pltpu.VMEM((NCHUNK, CH), jnp.float32),
            pltpu.VMEM((NCHK,), jnp.float32),
            pltpu.VMEM_SHARED((NPAD,), jnp.float32),
        ],
    )
    def deg_kernel(col_b, ew_b, deg_out, eidx, evalf, tmp, deg_sh):
        cid = lax.axis_index("c")
        sid = lax.axis_index("s")
        wid = sid * NC + cid
        nbase = sid * NCHK

        def _z(i, c):
            tmp[pl.ds(i * L, L)] = jnp.zeros((L,), jnp.float32)
            return c
        lax.fori_loop(0, NCHK // L, _z, 0)
        pltpu.sync_copy(tmp, deg_sh.at[pl.ds(nbase, NCHK)])
        plsc.subcore_barrier()

        pltpu.sync_copy(col_b.at[wid], eidx)
        pltpu.sync_copy(ew_b.at[wid], evalf)

        def _deg(j, c):
            pltpu.sync_copy(evalf.at[j], deg_sh.at[eidx.at[j]], add=True)
            return c
        lax.fori_loop(0, NCHUNK, _deg, 0)
        plsc.subcore_barrier()
        pltpu.sync_copy(deg_sh.at[pl.ds(nbase, NCHK)],
                        deg_out.at[cid, pl.ds(nbase, NCHK)])

    @functools.partial(
        pl.kernel,
        out_type=jax.ShapeDtypeStruct((NW, NCHUNK, CH), jnp.float32),
        mesh=mesh,
        compiler_params=pltpu.CompilerParams(needs_layout_passes=False),
        scratch_types=[
            pltpu.VMEM((NCHUNK, CH), jnp.int32),
            pltpu.VMEM((NCHUNK, CH), jnp.int32),
            pltpu.VMEM((NCHUNK, CH), jnp.float32),
            pltpu.VMEM((NCHUNK, CH), jnp.float32),
            pltpu.VMEM((NPAD,), jnp.float32),
        ],
    )
    def norm_kernel(row_b, col_b, ew_b, dinv, norm_out,
                    ridx, cidx, ewb, nbuf, dinv_loc):
        cid = lax.axis_index("c")
        sid = lax.axis_index("s")
        wid = sid * NC + cid

        pltpu.sync_copy(dinv, dinv_loc)
        pltpu.sync_copy(row_b.at[wid], ridx)
        pltpu.sync_copy(col_b.at[wid], cidx)
        pltpu.sync_copy(ew_b.at[wid], ewb)

        def _nrm(j, c):
            for k in range(CH // L):
                sl = pl.ds(k * L, L)
                ri = ridx[j, sl]
                ci = cidx[j, sl]
                w = ewb[j, sl]
                dv = plsc.load_gather(dinv_loc, [ri])
                dc = plsc.load_gather(dinv_loc, [ci])
                nbuf[j, sl] = dv * w * dc
            return c
        lax.fori_loop(0, NCHUNK, _nrm, 0)
        pltpu.sync_copy(nbuf, norm_out.at[wid])

    @functools.partial(
        pl.kernel,
        out_type=jax.ShapeDtypeStruct((NC, NPAD, F), jnp.float32),
        mesh=mesh,
        compiler_params=pltpu.CompilerParams(needs_layout_passes=False),
        scratch_types=[
            pltpu.VMEM((4, CH), jnp.int32),
            pltpu.VMEM((4, CH), jnp.int32),
            pltpu.VMEM((4, CH), jnp.float32),
            pltpu.VMEM((CH, F), jnp.float32),
            pltpu.VMEM((CH, F), jnp.float32),
            pltpu.VMEM_SHARED((NPAD, F), jnp.float32),
            pltpu.SemaphoreType.DMA,
            pltpu.SemaphoreType.DMA,
            pltpu.SemaphoreType.DMA,
            pltpu.SemaphoreType.DMA,
            pltpu.SemaphoreType.DMA,
            pltpu.SemaphoreType.DMA,
            pltpu.SemaphoreType.DMA,
            pltpu.SemaphoreType.DMA,
        ],
    )
    def scatter_kernel(h, row_f, col_f, norm_f, part,
                       ridxr, cidxr, nrmr, buf0, buf1, acc_sh,
                       sg0, sg1, si0, si1, si2, si3, ss0, ss1):
        cid = lax.axis_index("c")
        sid = lax.axis_index("s")
        nbase = sid * NCHK
        bufs = (buf0, buf1)
        sg = (sg0, sg1)
        si = (si0, si1, si2, si3)
        ss = (ss0, ss1)

        def _zr(r, c):
            for k in range(F // L):
                buf0[r, pl.ds(k * L, L)] = jnp.zeros((L,), jnp.float32)
            return c
        lax.fori_loop(0, CH, _zr, 0)
        for b in range(NCHK // CH):
            pltpu.sync_copy(buf0, acc_sh.at[pl.ds(nbase + b * CH, CH)])
        plsc.subcore_barrier()

        def fire_idx(q, s):
            pltpu.async_copy(row_f.at[q], ridxr.at[s], si[s])
            pltpu.async_copy(col_f.at[q], cidxr.at[s], si[s])
            pltpu.async_copy(norm_f.at[q], nrmr.at[s], si[s])

        def wait_idx(s):
            pltpu.make_async_copy(row_f.at[0], ridxr.at[s], si[s]).wait()
            pltpu.make_async_copy(col_f.at[0], cidxr.at[s], si[s]).wait()
            pltpu.make_async_copy(norm_f.at[0], nrmr.at[s], si[s]).wait()

        def fire_gather(rs, bs):
            pltpu.async_copy(h.at[ridxr.at[rs]], bufs[bs], sg[bs])

        def wait_gather(rs, bs):
            pltpu.make_async_copy(h.at[ridxr.at[rs]], bufs[bs],
                                  sg[bs]).wait()

        def fire_scatter(rs, bs):
            pltpu.async_copy(bufs[bs], acc_sh.at[cidxr.at[rs]], ss[bs],
                             add=True)

        def wait_scatter(rs, bs):
            pltpu.make_async_copy(bufs[bs], acc_sh.at[cidxr.at[rs]],
                                  ss[bs]).wait()

        def scale(rs, bs):
            r = bufs[bs]

            def _sce(e, c2):
                se = jnp.full((L,), rs, jnp.int32)
                ee = jnp.full((L,), e, jnp.int32)
                m = plsc.load_gather(nrmr, [se, ee])
                for k in range(F // L):
                    sl = pl.ds(k * L, L)
                    r[e, sl] = r[e, sl] * m
                return c2
            lax.fori_loop(0, CH, _sce, 0, unroll=2)

        def step(q, rs, wait_prev, fire2, fire1):
            bs = rs % 2
            nbs = 1 - bs
            wait_gather(rs, bs)
            if wait_prev:
                wait_scatter((rs + 3) % 4, nbs)
            if fire2:
                fire_idx(q + 2, (rs + 2) % 4)
            if fire1:
                wait_idx((rs + 1) % 4)
                fire_gather((rs + 1) % 4, nbs)
            scale(rs, bs)
            fire_scatter(rs, bs)

        def pipeline(base, n):
            fire_idx(base, 0)
            fire_idx(base + 1, 1)
            wait_idx(0)
            fire_gather(0, 0)
            step(base, 0, wait_prev=False, fire2=True, fire1=True)
            step(base + 1, 1, wait_prev=True, fire2=True, fire1=True)

            def _quad(jj, c):
                q = base + 4 * jj + 2
                step(q, 2, wait_prev=True, fire2=True, fire1=True)
                step(q + 1, 3, wait_prev=True, fire2=True, fire1=True)
                step(q + 2, 0, wait_prev=True, fire2=True, fire1=True)
                step(q + 3, 1, wait_prev=True, fire2=True, fire1=True)
                return c
            lax.fori_loop(0, (n - 4) // 4, _quad, 0)

            step(base + n - 2, 2, wait_prev=True, fire2=False, fire1=True)
            step(base + n - 1, 3, wait_prev=True, fire2=False, fire1=False)
            wait_scatter(3, 1)

        @pl.when(cid == 0)
        def _():
            pipeline(sid * NCH0, NCH0)

        @pl.when(cid == 1)
        def _():
            pipeline(NS * NCH0 + sid * NCH1, NCH1)

        plsc.subcore_barrier()

        pltpu.sync_copy(acc_sh.at[pl.ds(nbase, NCHK)],
                        part.at[cid, pl.ds(nbase, NCHK)])

    return deg_kernel, norm_kernel, scatter_kernel


_BLK = 640


def _dinv_body(dp_ref, dinv_ref, sn_ref):
    d = dp_ref[0] + dp_ref[1] + 1.0
    r = lax.rsqrt(d)
    dinv_ref[...] = r
    sn_ref[...] = r * r


def _tc_dinv(deg_parts):
    m = NPAD // F
    return pl.pallas_call(
        _dinv_body,
        out_shape=(
            jax.ShapeDtypeStruct((m, F), jnp.float32),
            jax.ShapeDtypeStruct((m, F), jnp.float32),
        ),
    )(deg_parts)


def _mm_body(x_ref, w_ref, o_ref):
    o_ref[...] = jnp.dot(x_ref[...], w_ref[...],
                         preferred_element_type=jnp.float32)


def _tc_matmul(x, w):
    return pl.pallas_call(
        _mm_body,
        grid=(NPAD // _BLK,),
        in_specs=[
            pl.BlockSpec((_BLK, F), lambda i: (i, 0)),
            pl.BlockSpec((F, F), lambda i: (0, 0)),
        ],
        out_specs=pl.BlockSpec((_BLK, F), lambda i: (i, 0)),
        out_shape=jax.ShapeDtypeStruct((NPAD, F), jnp.float32),
    )(x, w)


def _comb_body(p0_ref, p1_ref, hm_ref, sn_ref, b_ref, g_ref, bt_ref, w2_ref,
               h1_ref, h2m_ref):
    s = p0_ref[...] + p1_ref[...] + hm_ref[...] * sn_ref[...]
    h = jnp.maximum(s + b_ref[...], 0.0)
    h = h * (g_ref[...] * ISQ) + bt_ref[...]
    h1_ref[...] = h
    h2m_ref[...] = jnp.dot(h, w2_ref[...], preferred_element_type=jnp.float32)


def _tc_combine_mm(p0, p1, hm, sn, b, g, bt, w2):
    vec = pl.BlockSpec((1, F), lambda i: (0, 0))
    blk = pl.BlockSpec((_BLK, F), lambda i: (i, 0))
    col = pl.BlockSpec((_BLK, 1), lambda i: (i, 0))
    return pl.pallas_call(
        _comb_body,
        grid=(NPAD // _BLK,),
        in_specs=[blk, blk, blk, col, vec, vec, vec,
                  pl.BlockSpec((F, F), lambda i: (0, 0))],
        out_specs=(blk, blk),
        out_shape=(
            jax.ShapeDtypeStruct((NPAD, F), jnp.float32),
            jax.ShapeDtypeStruct((NPAD, F), jnp.float32),
        ),
    )(p0, p1, hm, sn, b.reshape(1, F), g.reshape(1, F), bt.reshape(1, F), w2)


def _head_body(p0_ref, p1_ref, hm_ref, sn_ref, b_ref, g_ref, bt_ref,
               x_ref, h1_ref,
               f1a_ref, f1b_ref, f1c_ref, fb1_ref, f2w_ref, f2b_ref, o_ref):
    s = p0_ref[...] + p1_ref[...] + hm_ref[...] * sn_ref[...]
    h2 = jnp.maximum(s + b_ref[...], 0.0)
    h2 = h2 * (g_ref[...] * ISQ) + bt_ref[...]
    z = (jnp.dot(x_ref[...], f1a_ref[...], preferred_element_type=jnp.float32)
         + jnp.dot(h1_ref[...], f1b_ref[...],
                   preferred_element_type=jnp.float32)
         + jnp.dot(h2, f1c_ref[...], preferred_element_type=jnp.float32))
    z = jnp.maximum(z + fb1_ref[...], 0.0)
    y = jnp.dot(z, f2w_ref[...], preferred_element_type=jnp.float32)
    o_ref[...] = jnp.maximum(y + f2b_ref[...], 0.0)


def _tc_head(p0, p1, hm, sn, b2, g2, bt2, x, h1, f1a, f1b, f1c, fb1,
             f2w, f2b):
    vec = pl.BlockSpec((1, F), lambda i: (0, 0))
    blk = pl.BlockSpec((_BLK, F), lambda i: (i, 0))
    col = pl.BlockSpec((_BLK, 1), lambda i: (i, 0))
    wspec = pl.BlockSpec((F, F), lambda i: (0, 0))
    return pl.pallas_call(
        _head_body,
        grid=(NPAD // _BLK,),
        in_specs=[blk, blk, blk, col, vec, vec, vec, blk, blk,
                  wspec, wspec, wspec,
                  vec, pl.BlockSpec((F, 1), lambda i: (0, 0)),
                  pl.BlockSpec((1, 1), lambda i: (0, 0))],
        out_specs=col,
        out_shape=jax.ShapeDtypeStruct((NPAD, 1), jnp.float32),
    )(p0, p1, hm, sn, b2.reshape(1, F), g2.reshape(1, F), bt2.reshape(1, F),
      x, h1, f1a, f1b, f1c, fb1.reshape(1, F), f2w, f2b.reshape(1, 1))


def kernel(x, adj_indices, adj_values, W1, b1, bn1_gamma, bn1_beta,
           W2, b2, bn2_gamma, bn2_beta, fc1_W, fc1_b, fc2_W, fc2_b):
    deg_kernel, norm_kernel, scatter_kernel = _sc_kernels()
    ipad = jnp.zeros((EP - E,), jnp.int32)
    row_p = jnp.concatenate([adj_indices[0], ipad])
    col_p = jnp.concatenate([adj_indices[1], ipad])
    ew_p = jnp.concatenate([adj_values, jnp.zeros((EP - E,), jnp.float32)])
    row_b = row_p.reshape(NW, NCHUNK, CH)
    col_b = col_p.reshape(NW, NCHUNK, CH)
    ew_b = ew_p.reshape(NW, NCHUNK, CH)
    row_f = row_p.reshape(TOTCH, CH)
    col_f = col_p.reshape(TOTCH, CH)
    x_pad = jnp.concatenate([x, jnp.zeros((NPAD - N, F), jnp.float32)])

    deg_parts = deg_kernel(col_b, ew_b)
    dinv2d, sn2d = _tc_dinv(deg_parts.reshape(NC, NPAD // F, F))
    norm_b = norm_kernel(row_b, col_b, ew_b, dinv2d.reshape(NPAD))
    norm_f = norm_b.reshape(TOTCH, CH)
    sn = sn2d.reshape(NPAD, 1)

    h1m = _tc_matmul(x_pad, W1)
    part1 = scatter_kernel(h1m, row_f, col_f, norm_f)
    h1, h2m = _tc_combine_mm(part1[0], part1[1], h1m, sn,
                             b1, bn1_gamma, bn1_beta, W2)
    part2 = scatter_kernel(h2m, row_f, col_f, norm_f)
    y = _tc_head(part2[0], part2[1], h2m, sn, b2, bn2_gamma, bn2_beta,
                 x_pad, h1, fc1_W[:F], fc1_W[F:2 * F], fc1_W[2 * F:],
                 fc1_b, fc2_W, fc2_b)
    return y.reshape(-1)[:N]
